# baseline (speedup 1.0000x reference)
import functools

import jax
import jax.numpy as jnp
from jax import lax
from jax.experimental import pallas as pl
from jax.experimental.pallas import tpu as pltpu
from jax.experimental.pallas import tpu_sc as plsc

BATCH = 16384
HIST = 200
EMBED_DIM = 64
LANES = 16
BB = 128
NUM_BLOCKS = BATCH // BB
NUM_WORKERS = 32
BLOCKS_PER_W = NUM_BLOCKS // NUM_WORKERS
NUM_PAIRS = HIST // 2


def _gather_transposed(table, idx_t):
    out_shape = (HIST * 8, NUM_BLOCKS, 8 * BB)

    @functools.partial(
        pl.kernel,
        out_type=jax.ShapeDtypeStruct(out_shape, jnp.float32),
        mesh=plsc.VectorSubcoreMesh(
            core_axis_name="c", subcore_axis_name="s"
        ),
        scratch_types=[
            pltpu.VMEM((HIST, BB), jnp.int32),
            pltpu.VMEM((BB, EMBED_DIM), jnp.float32),
            pltpu.VMEM((BB, EMBED_DIM), jnp.float32),
            pltpu.VMEM((8, 8 * BB), jnp.float32),
            pltpu.VMEM((8, 8 * BB), jnp.float32),
            pltpu.SemaphoreType.DMA,
            pltpu.SemaphoreType.DMA,
            pltpu.SemaphoreType.DMA,
            pltpu.SemaphoreType.DMA,
        ],
        compiler_params=pltpu.CompilerParams(
            use_tc_tiling_on_sc=False, needs_layout_passes=False
        ),
    )
    def k(table_hbm, idx_hbm, out_hbm,
          idxb, gbuf0, gbuf1, tbuf0, tbuf1, gsem0, gsem1, ssem0, ssem1):
        wid = lax.axis_index("s") * 2 + lax.axis_index("c")

        def fire_gather(h, gbuf, sem):
            pltpu.async_copy(table_hbm.at[idxb.at[h]], gbuf, sem)

        def gather_wait(h, gbuf, sem):
            pltpu.make_async_copy(
                table_hbm.at[idxb.at[h]], gbuf, sem
            ).wait()

        def fire_store(h, bt, tbuf, sem):
            pltpu.async_copy(
                tbuf, out_hbm.at[pl.ds(h * 8, 8), bt], sem
            )

        def store_wait(h, bt, tbuf, sem):
            pltpu.make_async_copy(
                tbuf, out_hbm.at[pl.ds(h * 8, 8), bt], sem
            ).wait()

        rowv = [
            lax.iota(jnp.int32, LANES) + (bj * LANES) for bj in range(8)
        ]

        def transpose(gbuf, tbuf):
            for dt in range(8):
                for di in range(8):
                    d = dt * 8 + di
                    colv = jnp.full((LANES,), d, jnp.int32)
                    for bj in range(8):
                        v = plsc.load_gather(gbuf, [rowv[bj], colv])
                        tbuf[dt, pl.ds(di * BB + bj * LANES, LANES)] = v

        def block_body(blk, _):
            bt = wid * BLOCKS_PER_W + blk
            pltpu.sync_copy(
                idx_hbm.at[pl.ds(0, HIST), pl.ds(bt * BB, BB)], idxb
            )
            fire_gather(0, gbuf0, gsem0)

            def pair_body(p, _):
                h0 = 2 * p
                h1 = h0 + 1

                fire_gather(h1, gbuf1, gsem1)
                gather_wait(h0, gbuf0, gsem0)

                @pl.when(p > 0)
                def _():
                    store_wait(h0 - 2, bt, tbuf0, ssem0)

                transpose(gbuf0, tbuf0)
                fire_store(h0, bt, tbuf0, ssem0)

                @pl.when(p < NUM_PAIRS - 1)
                def _():
                    fire_gather(h0 + 2, gbuf0, gsem0)

                gather_wait(h1, gbuf1, gsem1)

                @pl.when(p > 0)
                def _():
                    store_wait(h1 - 2, bt, tbuf1, ssem1)

                transpose(gbuf1, tbuf1)
                fire_store(h1, bt, tbuf1, ssem1)
                return 0

            lax.fori_loop(0, NUM_PAIRS, pair_body, 0)

            store_wait(HIST - 2, bt, tbuf0, ssem0)
            store_wait(HIST - 1, bt, tbuf1, ssem1)
            return 0

        lax.fori_loop(0, BLOCKS_PER_W, block_body, 0)

    return k(table, idx_t)


def kernel(x, table):
    idx_t = x.T.astype(jnp.int32)
    out3 = _gather_transposed(table, idx_t)
    out6 = out3.reshape(HIST, 8, NUM_BLOCKS, 8, BB)
    return jnp.transpose(out6, (2, 4, 0, 1, 3)).reshape(BATCH, HIST, EMBED_DIM)


# speedup vs baseline: 4.1233x; 4.1233x over previous
import functools

import jax
import jax.numpy as jnp
from jax import lax
from jax.experimental import pallas as pl
from jax.experimental.pallas import tpu as pltpu
from jax.experimental.pallas import tpu_sc as plsc

BATCH = 16384
HIST = 200
EMBED_DIM = 64
PADDED_DIM = 128
NUM_WORKERS = 32
GB = 4
GROUP = GB * HIST
CHUNKS = [(i * 128, 128) for i in range(GROUP // 128)]
if GROUP % 128:
    CHUNKS.append((GROUP - GROUP % 128, GROUP % 128))


def _gather_rows(table, idx):
    batches_per_w = BATCH // NUM_WORKERS
    num_groups = batches_per_w // GB
    num_pairs = num_groups // 2

    @functools.partial(
        pl.kernel,
        out_type=jax.ShapeDtypeStruct((BATCH, HIST, PADDED_DIM), jnp.float32),
        mesh=plsc.VectorSubcoreMesh(
            core_axis_name="c", subcore_axis_name="s"
        ),
        scratch_types=[
            pltpu.VMEM((GROUP,), jnp.int32),
            pltpu.VMEM((GROUP,), jnp.int32),
            pltpu.VMEM((GROUP, EMBED_DIM), jnp.float32),
            pltpu.VMEM((GROUP, EMBED_DIM), jnp.float32),
            pltpu.SemaphoreType.DMA,
            pltpu.SemaphoreType.DMA,
            pltpu.SemaphoreType.DMA,
            pltpu.SemaphoreType.DMA,
        ],
        compiler_params=pltpu.CompilerParams(use_tc_tiling_on_sc=False),
    )
    def k(table_hbm, idx_hbm, out_hbm,
          idx0, idx1, rows0, rows1, gsem0, gsem1, ssem0, ssem1):
        wid = lax.axis_index("s") * 2 + lax.axis_index("c")
        base = wid * batches_per_w

        def fire_gathers(idx_v, rows_v, sem, gbatch):
            pltpu.sync_copy(idx_hbm.at[pl.ds(gbatch * HIST, GROUP)], idx_v)
            for r, n in CHUNKS:
                pltpu.async_copy(
                    table_hbm.at[idx_v.at[pl.ds(r, n)]],
                    rows_v.at[pl.ds(r, n)],
                    sem,
                )

        def drain_gathers(idx_v, rows_v, sem):
            for r, n in CHUNKS:
                pltpu.make_async_copy(
                    table_hbm.at[idx_v.at[pl.ds(r, n)]],
                    rows_v.at[pl.ds(r, n)],
                    sem,
                ).wait()

        def fire_stores(rows_v, sem, gbatch):
            for b in range(GB):
                pltpu.async_copy(
                    rows_v.at[pl.ds(b * HIST, HIST)],
                    out_hbm.at[gbatch + b, pl.ds(0, HIST), pl.ds(0, EMBED_DIM)],
                    sem,
                )

        def store_wait(rows_v, sem, gbatch):
            for b in range(GB):
                pltpu.make_async_copy(
                    rows_v.at[pl.ds(b * HIST, HIST)],
                    out_hbm.at[gbatch + b, pl.ds(0, HIST), pl.ds(0, EMBED_DIM)],
                    sem,
                ).wait()

        fire_gathers(idx0, rows0, gsem0, base)

        def body(h, _):
            g0 = base + (2 * h) * GB
            g1 = g0 + GB
            g2 = g1 + GB

            @pl.when(h > 0)
            def _():
                store_wait(rows1, ssem1, g1 - 2 * GB)

            fire_gathers(idx1, rows1, gsem1, g1)

            drain_gathers(idx0, rows0, gsem0)
            fire_stores(rows0, ssem0, g0)

            @pl.when(h < num_pairs - 1)
            def _():
                store_wait(rows0, ssem0, g0)
                fire_gathers(idx0, rows0, gsem0, g2)

            drain_gathers(idx1, rows1, gsem1)
            fire_stores(rows1, ssem1, g1)
            return 0

        lax.fori_loop(0, num_pairs, body, 0)

        last = base + (num_groups - 2) * GB
        store_wait(rows0, ssem0, last)
        store_wait(rows1, ssem1, last + GB)

    return k(table, idx)


def kernel(x, table):
    idx_flat = x.reshape(-1).astype(jnp.int32)
    out = _gather_rows(table, idx_flat)
    return out[:, :, :EMBED_DIM]


# speedup vs baseline: 4.1743x; 1.0124x over previous
import functools

import jax
import jax.numpy as jnp
from jax import lax
from jax.experimental import pallas as pl
from jax.experimental.pallas import tpu as pltpu
from jax.experimental.pallas import tpu_sc as plsc

BATCH = 16384
HIST = 200
EMBED_DIM = 64
PADDED_DIM = 128
NUM_WORKERS = 32
GB = 4
GROUP = GB * HIST
CHUNKS = [(i * 128, 128) for i in range(GROUP // 128)]
if GROUP % 128:
    CHUNKS.append((GROUP - GROUP % 128, GROUP % 128))


def _gather_rows(table, idx):
    batches_per_w = BATCH // NUM_WORKERS
    num_groups = batches_per_w // GB
    num_pairs = num_groups // 2

    @functools.partial(
        pl.kernel,
        out_type=jax.ShapeDtypeStruct((BATCH, HIST, PADDED_DIM), jnp.float32),
        mesh=plsc.VectorSubcoreMesh(
            core_axis_name="c", subcore_axis_name="s"
        ),
        scratch_types=[
            pltpu.VMEM((GROUP,), jnp.int32),
            pltpu.VMEM((GROUP,), jnp.int32),
            pltpu.VMEM((GROUP, EMBED_DIM), jnp.float32),
            pltpu.VMEM((GROUP, EMBED_DIM), jnp.float32),
            pltpu.SemaphoreType.DMA,
            pltpu.SemaphoreType.DMA,
            pltpu.SemaphoreType.DMA,
            pltpu.SemaphoreType.DMA,
            pltpu.SemaphoreType.DMA,
            pltpu.SemaphoreType.DMA,
        ],
        compiler_params=pltpu.CompilerParams(use_tc_tiling_on_sc=False),
    )
    def k(table_hbm, idx_hbm, out_hbm,
          idx0, idx1, rows0, rows1,
          gsem0, gsem1, ssem0, ssem1, isem0, isem1):
        wid = lax.axis_index("s") * 2 + lax.axis_index("c")
        base = wid * batches_per_w

        def fire_idx(idx_v, sem, gbatch):
            pltpu.async_copy(
                idx_hbm.at[pl.ds(gbatch * HIST, GROUP)], idx_v, sem
            )

        def idx_wait(idx_v, sem, gbatch):
            pltpu.make_async_copy(
                idx_hbm.at[pl.ds(gbatch * HIST, GROUP)], idx_v, sem
            ).wait()

        def fire_gathers(idx_v, rows_v, sem):
            for r, n in CHUNKS:
                pltpu.async_copy(
                    table_hbm.at[idx_v.at[pl.ds(r, n)]],
                    rows_v.at[pl.ds(r, n)],
                    sem,
                )

        def drain_gathers(idx_v, rows_v, sem):
            for r, n in CHUNKS:
                pltpu.make_async_copy(
                    table_hbm.at[idx_v.at[pl.ds(r, n)]],
                    rows_v.at[pl.ds(r, n)],
                    sem,
                ).wait()

        def fire_stores(rows_v, sem, gbatch):
            for b in range(GB):
                pltpu.async_copy(
                    rows_v.at[pl.ds(b * HIST, HIST)],
                    out_hbm.at[gbatch + b, pl.ds(0, HIST), pl.ds(0, EMBED_DIM)],
                    sem,
                )

        def store_wait(rows_v, sem, gbatch):
            for b in range(GB):
                pltpu.make_async_copy(
                    rows_v.at[pl.ds(b * HIST, HIST)],
                    out_hbm.at[gbatch + b, pl.ds(0, HIST), pl.ds(0, EMBED_DIM)],
                    sem,
                ).wait()

        fire_idx(idx0, isem0, base)
        fire_idx(idx1, isem1, base + GB)
        idx_wait(idx0, isem0, base)
        fire_gathers(idx0, rows0, gsem0)

        def body(h, _):
            g0 = base + (2 * h) * GB
            g1 = g0 + GB
            g2 = g1 + GB
            g3 = g2 + GB

            @pl.when(h > 0)
            def _():
                store_wait(rows1, ssem1, g1 - 2 * GB)

            idx_wait(idx1, isem1, g1)
            fire_gathers(idx1, rows1, gsem1)

            drain_gathers(idx0, rows0, gsem0)
            fire_stores(rows0, ssem0, g0)

            @pl.when(h < num_pairs - 1)
            def _():
                fire_idx(idx0, isem0, g2)
                store_wait(rows0, ssem0, g0)
                idx_wait(idx0, isem0, g2)
                fire_gathers(idx0, rows0, gsem0)

            drain_gathers(idx1, rows1, gsem1)

            @pl.when(h < num_pairs - 1)
            def _():
                fire_idx(idx1, isem1, g3)

            fire_stores(rows1, ssem1, g1)
            return 0

        lax.fori_loop(0, num_pairs, body, 0)

        last = base + (num_groups - 2) * GB
        store_wait(rows0, ssem0, last)
        store_wait(rows1, ssem1, last + GB)

    return k(table, idx)


def kernel(x, table):
    idx_flat = x.reshape(-1).astype(jnp.int32)
    out = _gather_rows(table, idx_flat)
    return out[:, :, :EMBED_DIM]


# speedup vs baseline: 4.1763x; 1.0005x over previous
import functools

import jax
import jax.numpy as jnp
from jax import lax
from jax.experimental import pallas as pl
from jax.experimental.pallas import tpu as pltpu
from jax.experimental.pallas import tpu_sc as plsc

BATCH = 16384
HIST = 200
EMBED_DIM = 64
PADDED_DIM = 128
NUM_WORKERS = 32
GB = 4
GROUP = GB * HIST
CHUNKS = [(i * 128, 128) for i in range(GROUP // 128)]
if GROUP % 128:
    CHUNKS.append((GROUP - GROUP % 128, GROUP % 128))


def _gather_rows(table, idx):
    batches_per_w = BATCH // NUM_WORKERS
    num_groups = batches_per_w // GB
    num_pairs = num_groups // 2

    @functools.partial(
        pl.kernel,
        out_type=jax.ShapeDtypeStruct((BATCH, HIST, PADDED_DIM), jnp.float32),
        mesh=plsc.VectorSubcoreMesh(
            core_axis_name="c", subcore_axis_name="s"
        ),
        scratch_types=[
            pltpu.VMEM((GROUP,), jnp.int32),
            pltpu.VMEM((GROUP,), jnp.int32),
            pltpu.VMEM((GB, HIST, EMBED_DIM), jnp.float32),
            pltpu.VMEM((GB, HIST, EMBED_DIM), jnp.float32),
            pltpu.SemaphoreType.DMA,
            pltpu.SemaphoreType.DMA,
            pltpu.SemaphoreType.DMA,
            pltpu.SemaphoreType.DMA,
            pltpu.SemaphoreType.DMA,
            pltpu.SemaphoreType.DMA,
        ],
        compiler_params=pltpu.CompilerParams(use_tc_tiling_on_sc=False),
    )
    def k(table_hbm, idx_hbm, out_hbm,
          idx0, idx1, rows0, rows1,
          gsem0, gsem1, ssem0, ssem1, isem0, isem1):
        wid = lax.axis_index("s") * 2 + lax.axis_index("c")
        base = wid * batches_per_w

        def fire_idx(idx_v, sem, gbatch):
            pltpu.async_copy(
                idx_hbm.at[pl.ds(gbatch * HIST, GROUP)], idx_v, sem
            )

        def idx_wait(idx_v, sem, gbatch):
            pltpu.make_async_copy(
                idx_hbm.at[pl.ds(gbatch * HIST, GROUP)], idx_v, sem
            ).wait()

        def fire_gathers(idx_v, rows_v, sem):
            for b in range(GB):
                for r, n in ((0, 128), (128, 72)):
                    pltpu.async_copy(
                        table_hbm.at[idx_v.at[pl.ds(b * HIST + r, n)]],
                        rows_v.at[b, pl.ds(r, n)],
                        sem,
                    )

        def drain_gathers(idx_v, rows_v, sem):
            for b in range(GB):
                for r, n in ((0, 128), (128, 72)):
                    pltpu.make_async_copy(
                        table_hbm.at[idx_v.at[pl.ds(b * HIST + r, n)]],
                        rows_v.at[b, pl.ds(r, n)],
                        sem,
                    ).wait()

        def fire_stores(rows_v, sem, gbatch):
            pltpu.async_copy(
                rows_v,
                out_hbm.at[pl.ds(gbatch, GB), pl.ds(0, HIST),
                           pl.ds(0, EMBED_DIM)],
                sem,
            )

        def store_wait(rows_v, sem, gbatch):
            pltpu.make_async_copy(
                rows_v,
                out_hbm.at[pl.ds(gbatch, GB), pl.ds(0, HIST),
                           pl.ds(0, EMBED_DIM)],
                sem,
            ).wait()

        fire_idx(idx0, isem0, base)
        fire_idx(idx1, isem1, base + GB)
        idx_wait(idx0, isem0, base)
        fire_gathers(idx0, rows0, gsem0)

        def body(h, _):
            g0 = base + (2 * h) * GB
            g1 = g0 + GB
            g2 = g1 + GB
            g3 = g2 + GB

            @pl.when(h > 0)
            def _():
                store_wait(rows1, ssem1, g1 - 2 * GB)

            idx_wait(idx1, isem1, g1)
            fire_gathers(idx1, rows1, gsem1)

            drain_gathers(idx0, rows0, gsem0)
            fire_stores(rows0, ssem0, g0)

            @pl.when(h < num_pairs - 1)
            def _():
                fire_idx(idx0, isem0, g2)
                store_wait(rows0, ssem0, g0)
                idx_wait(idx0, isem0, g2)
                fire_gathers(idx0, rows0, gsem0)

            drain_gathers(idx1, rows1, gsem1)

            @pl.when(h < num_pairs - 1)
            def _():
                fire_idx(idx1, isem1, g3)

            fire_stores(rows1, ssem1, g1)
            return 0

        lax.fori_loop(0, num_pairs, body, 0)

        last = base + (num_groups - 2) * GB
        store_wait(rows0, ssem0, last)
        store_wait(rows1, ssem1, last + GB)

    return k(table, idx)


def kernel(x, table):
    idx_flat = x.reshape(-1).astype(jnp.int32)
    out = _gather_rows(table, idx_flat)
    return out[:, :, :EMBED_DIM]
